# Initial kernel scaffold; baseline (speedup 1.0000x reference)
import jax
import jax.numpy as jnp
from jax.experimental import pallas as pl


def kernel(input, another_input, table1, table2):
    raise NotImplementedError("write your pallas kernel here")


# speedup vs baseline: 1.4349x; 1.4349x over previous
import jax
import jax.numpy as jnp
from jax import lax
from jax.experimental import pallas as pl
from jax.experimental.pallas import tpu as pltpu
from jax.experimental.pallas import tpu_sc as plsc

NUM_EMB = 1_000_000
DIM = 32
BATCH = 16384
HIST = 20
N = BATCH * HIST

NC = 2
NS = 16
NW = NC * NS

PER_W = N // NW
SUB = 128
ROWS_PER_W = PER_W // SUB
CHUNK = 1024
SUB_PER_CHUNK = CHUNK // SUB
NCHUNK = PER_W // CHUNK


def _body(idx1_hbm, idx2_hbm, t1_hbm, t2_hbm, out_hbm,
          idx1_v, idx2_v, rows1_v, rows2_v, sem):
    c = lax.axis_index("c")
    s = lax.axis_index("s")
    wid = s * NC + c
    pltpu.sync_copy(idx1_hbm.at[wid], idx1_v)
    pltpu.sync_copy(idx2_hbm.at[wid], idx2_v)
    base = wid * PER_W

    def chunk_body(ci, _):
        descs = []
        for j in range(SUB_PER_CHUNK):
            row = ci * SUB_PER_CHUNK + j
            dst = pl.ds(j * SUB, SUB)
            descs.append(pltpu.async_copy(
                t1_hbm.at[idx1_v.at[row]], rows1_v.at[dst], sem))
            descs.append(pltpu.async_copy(
                t2_hbm.at[idx2_v.at[row]], rows2_v.at[dst], sem))
        for d in descs:
            d.wait()

        def add_body(i, _):
            lo = pl.ds(0, 16)
            hi = pl.ds(16, 16)
            rows1_v[i, lo] = rows1_v[i, lo] + rows2_v[i, lo]
            rows1_v[i, hi] = rows1_v[i, hi] + rows2_v[i, hi]
            return ()

        lax.fori_loop(0, CHUNK, add_body, ())
        pltpu.sync_copy(rows1_v, out_hbm.at[pl.ds(base + ci * CHUNK, CHUNK)])
        return ()

    lax.fori_loop(0, NCHUNK, chunk_body, ())


def kernel(input, another_input, table1, table2):
    idx1 = input.reshape(-1).astype(jnp.int32).reshape(NW, ROWS_PER_W, SUB)
    idx2 = another_input.reshape(-1).astype(jnp.int32).reshape(NW, ROWS_PER_W, SUB)
    mesh = plsc.VectorSubcoreMesh(core_axis_name="c", subcore_axis_name="s")
    out = pl.kernel(
        _body,
        out_type=jax.ShapeDtypeStruct((N, DIM), jnp.float32),
        mesh=mesh,
        compiler_params=pltpu.CompilerParams(use_tc_tiling_on_sc=False),
        scratch_types=[
            pltpu.VMEM((ROWS_PER_W, SUB), jnp.int32),
            pltpu.VMEM((ROWS_PER_W, SUB), jnp.int32),
            pltpu.VMEM((CHUNK, DIM), jnp.float32),
            pltpu.VMEM((CHUNK, DIM), jnp.float32),
            pltpu.SemaphoreType.DMA,
        ],
    )(idx1, idx2, table1, table2)
    return out.reshape(BATCH, HIST, DIM)


# speedup vs baseline: 1.4937x; 1.0410x over previous
import jax
import jax.numpy as jnp
from jax import lax
from jax.experimental import pallas as pl
from jax.experimental.pallas import tpu as pltpu
from jax.experimental.pallas import tpu_sc as plsc

DIM = 32
BATCH = 16384
HIST = 20
N = BATCH * HIST

NC = 2
NS = 16
NW = NC * NS

PER_W = N // NW
SUB = 128
ROWS_PER_W = PER_W // SUB
CHUNK = 2048
SUB_PER_CHUNK = CHUNK // SUB
NCHUNK = PER_W // CHUNK


def _body(idx1_hbm, idx2_hbm, t1_hbm, t2_hbm, out_hbm,
          idx1_v, idx2_v, rows_v, sem):
    c = lax.axis_index("c")
    s = lax.axis_index("s")
    wid = s * NC + c
    pltpu.sync_copy(idx1_hbm.at[wid], idx1_v)
    pltpu.sync_copy(idx2_hbm.at[wid], idx2_v)
    base = wid * PER_W

    def chunk_body(ci, _):
        descs = []
        for j in range(SUB_PER_CHUNK):
            row = ci * SUB_PER_CHUNK + j
            dst = pl.ds(j * SUB, SUB)
            descs.append(pltpu.async_copy(
                t1_hbm.at[idx1_v.at[row]], rows_v.at[dst], sem))
        for d in descs:
            d.wait()
        descs = []
        for j in range(SUB_PER_CHUNK):
            row = ci * SUB_PER_CHUNK + j
            dst = pl.ds(j * SUB, SUB)
            descs.append(pltpu.async_copy(
                t2_hbm.at[idx2_v.at[row]], rows_v.at[dst], sem, add=True))
        for d in descs:
            d.wait()
        pltpu.sync_copy(rows_v, out_hbm.at[pl.ds(base + ci * CHUNK, CHUNK)])
        return ()

    lax.fori_loop(0, NCHUNK, chunk_body, ())


def kernel(input, another_input, table1, table2):
    idx1 = input.reshape(-1).astype(jnp.int32).reshape(NW, ROWS_PER_W, SUB)
    idx2 = another_input.reshape(-1).astype(jnp.int32).reshape(NW, ROWS_PER_W, SUB)
    mesh = plsc.VectorSubcoreMesh(core_axis_name="c", subcore_axis_name="s")
    out = pl.kernel(
        _body,
        out_type=jax.ShapeDtypeStruct((N, DIM), jnp.float32),
        mesh=mesh,
        compiler_params=pltpu.CompilerParams(use_tc_tiling_on_sc=False),
        scratch_types=[
            pltpu.VMEM((ROWS_PER_W, SUB), jnp.int32),
            pltpu.VMEM((ROWS_PER_W, SUB), jnp.int32),
            pltpu.VMEM((CHUNK, DIM), jnp.float32),
            pltpu.SemaphoreType.DMA,
        ],
    )(idx1, idx2, table1, table2)
    return out.reshape(BATCH, HIST, DIM)
